# Initial kernel scaffold; baseline (speedup 1.0000x reference)
import jax
import jax.numpy as jnp
from jax.experimental import pallas as pl


def kernel(target_deltas, target_scores, output_deltas, output_scores):
    raise NotImplementedError("write your pallas kernel here")


# speedup vs baseline: 1.5263x; 1.5263x over previous
import jax
import jax.numpy as jnp
from jax.experimental import pallas as pl
from jax.experimental.pallas import tpu as pltpu

N = 49152
ROWS = N // 128


def _loss_body(ts_ref, os_ref, td_ref, od_ref, out_ref):
    ts = ts_ref[...]
    os_ = os_ref[...]
    td = td_ref[...]
    od = od_ref[...]

    valid = jnp.not_equal(ts, -1.0)
    validf = valid.astype(jnp.float32)

    eps = 1e-7
    p = jnp.clip(os_, eps, 1.0 - eps)
    bce = -(ts * jnp.log(p) + (1.0 - ts) * jnp.log(1.0 - p))
    bce_sum = jnp.sum(jnp.where(valid, bce, 0.0))
    vcount = jnp.sum(validf)

    p_star = jnp.where(ts > 0.0, 1.0, 0.0) * validf
    d = jnp.abs(od - td)
    sl1 = jnp.where(d < 1.0, 0.5 * d * d, d - 0.5)
    a_y = jnp.sum(sl1, axis=0)
    reg_sum = jnp.sum(p_star * a_y)
    pcount = jnp.sum(p_star)

    a = bce_sum / jnp.maximum(vcount, 1.0)
    b = reg_sum / jnp.maximum(1e-7, pcount)
    out_ref[0, 0] = a + b


def kernel(target_deltas, target_scores, output_deltas, output_scores):
    ts = target_scores.reshape(ROWS, 128)
    os_ = output_scores.reshape(ROWS, 128)
    td = target_deltas.reshape(N, 4).T.reshape(4, ROWS, 128)
    od = output_deltas.reshape(N, 4).T.reshape(4, ROWS, 128)

    out = pl.pallas_call(
        _loss_body,
        out_shape=jax.ShapeDtypeStruct((1, 1), jnp.float32),
        out_specs=pl.BlockSpec(memory_space=pltpu.SMEM),
    )(ts, os_, td, od)
    return out[0, 0]
